# Initial kernel scaffold; baseline (speedup 1.0000x reference)
import jax
import jax.numpy as jnp
from jax.experimental import pallas as pl


def kernel(features_table, neighbors, pick):
    raise NotImplementedError("write your pallas kernel here")


# speedup vs baseline: 1.0459x; 1.0459x over previous
import functools

import jax
import jax.numpy as jnp
from jax import lax
from jax.experimental import pallas as pl
from jax.experimental.pallas import tpu as pltpu
from jax.experimental.pallas import tpu_sc as plsc

N_NODES = 100000
D_FEAT = 128
B = 100000
K = 16

NC = 2
NS = 16
NW = NC * NS

CHUNK = 3200
SUB = 128
NSUB = CHUNK // SUB
LAST_BASE = B - CHUNK


def _body(table_hbm, neigh_hbm, pick_hbm, out_hbm,
          pick_v, neigh_v, chosen_v, rows_v, sem):
    wid = lax.axis_index("s") * NC + lax.axis_index("c")
    base = jnp.minimum(wid * CHUNK, LAST_BASE)

    pltpu.sync_copy(pick_hbm.at[pl.ds(base, CHUNK)], pick_v)
    pltpu.sync_copy(neigh_hbm.at[pl.ds(base * K, CHUNK * K)], neigh_v)

    lane = lax.iota(jnp.int32, 16)

    def resolve(r, _):
        for c8 in range(8):
            i = r * 8 + c8
            pick16 = pick_v[pl.ds(i * 16, 16)]
            flat16 = (i * 16 + lane) * K + pick16
            chosen16 = plsc.load_gather(neigh_v, [flat16])
            chosen_v[r, pl.ds(c8 * 16, 16)] = chosen16
        return ()

    lax.fori_loop(0, NSUB, resolve, (), unroll=False)

    def gather(j, _):
        cp = pltpu.make_async_copy(table_hbm.at[chosen_v.at[j]], rows_v, sem)
        cp.start()
        cp.wait()
        pltpu.sync_copy(rows_v, out_hbm.at[pl.ds(base + j * SUB, SUB), :])
        return ()

    lax.fori_loop(0, NSUB, gather, (), unroll=False)


@jax.jit
def kernel(features_table, neighbors, pick):
    mesh = plsc.VectorSubcoreMesh(core_axis_name="c", subcore_axis_name="s")
    run = pl.kernel(
        _body,
        out_type=jax.ShapeDtypeStruct((B, D_FEAT), jnp.float32),
        mesh=mesh,
        scratch_types=[
            pltpu.VMEM((CHUNK,), jnp.int32),
            pltpu.VMEM((CHUNK * K,), jnp.int32),
            pltpu.VMEM((NSUB, SUB), jnp.int32),
            pltpu.VMEM((SUB, D_FEAT), jnp.float32),
            pltpu.SemaphoreType.DMA,
        ],
        compiler_params=pltpu.CompilerParams(needs_layout_passes=False),
    )
    return run(features_table, neighbors.reshape(-1), pick)


# speedup vs baseline: 1.2073x; 1.1543x over previous
import jax
import jax.numpy as jnp
from jax import lax
from jax.experimental import pallas as pl
from jax.experimental.pallas import tpu as pltpu
from jax.experimental.pallas import tpu_sc as plsc

N_NODES = 100000
D_FEAT = 128
B = 100000
K = 16

NC = 2
NS = 16
NW = NC * NS

CHUNK = 3200
SUB = 80
NSUB = CHUNK // SUB
NBUF = 5
NGRP = NSUB // NBUF
LAST_BASE = B - CHUNK


def _body(table_hbm, neigh_hbm, pick_hbm, out_hbm,
          pick_v, neigh_v, chosen_v, rows_v, sem_g, sem_s):
    wid = lax.axis_index("s") * NC + lax.axis_index("c")
    base = jnp.minimum(wid * CHUNK, LAST_BASE)

    pltpu.sync_copy(pick_hbm.at[pl.ds(base, CHUNK)], pick_v)
    pltpu.sync_copy(neigh_hbm.at[pl.ds(base * K, CHUNK * K)], neigh_v)

    lane = lax.iota(jnp.int32, 16)

    def resolve(r, _):
        for c8 in range(4):
            i = r * 4 + c8
            pick16 = pick_v[pl.ds(i * 16, 16)]
            flat16 = (i * 16 + lane) * K + pick16
            chosen16 = plsc.load_gather(neigh_v, [flat16])
            chosen_v[pl.ds(i * 16, 16)] = chosen16
        return ()

    lax.fori_loop(0, CHUNK // 64, resolve, (), unroll=False)

    def gather_start(j, slot):
        pltpu.make_async_copy(
            table_hbm.at[chosen_v.at[pl.ds(j * SUB, SUB)]],
            rows_v.at[slot], sem_g.at[slot]
        ).start()

    def gather_wait(j, slot):
        pltpu.make_async_copy(
            table_hbm.at[chosen_v.at[pl.ds(j * SUB, SUB)]],
            rows_v.at[slot], sem_g.at[slot]
        ).wait()

    def store_start(j, slot):
        pltpu.make_async_copy(
            rows_v.at[slot], out_hbm.at[pl.ds(base + j * SUB, SUB), :],
            sem_s.at[slot],
        ).start()

    def store_wait(j, slot):
        pltpu.make_async_copy(
            rows_v.at[slot], out_hbm.at[pl.ds(base + j * SUB, SUB), :],
            sem_s.at[slot],
        ).wait()

    for b in range(NBUF):
        gather_start(b, b)

    def group(g, _):
        for b in range(NBUF):
            j = g * NBUF + b
            gather_wait(j, b)
            store_start(j, b)
        for b in range(NBUF):
            j = g * NBUF + b
            store_wait(j, b)

            @pl.when(j + NBUF < NSUB)
            def _():
                gather_start(j + NBUF, b)
        return ()

    lax.fori_loop(0, NGRP, group, (), unroll=False)


@jax.jit
def kernel(features_table, neighbors, pick):
    mesh = plsc.VectorSubcoreMesh(core_axis_name="c", subcore_axis_name="s")
    run = pl.kernel(
        _body,
        out_type=jax.ShapeDtypeStruct((B, D_FEAT), jnp.float32),
        mesh=mesh,
        scratch_types=[
            pltpu.VMEM((CHUNK,), jnp.int32),
            pltpu.VMEM((CHUNK * K,), jnp.int32),
            pltpu.VMEM((CHUNK,), jnp.int32),
            pltpu.VMEM((NBUF, SUB, D_FEAT), jnp.float32),
            pltpu.SemaphoreType.DMA((NBUF,)),
            pltpu.SemaphoreType.DMA((NBUF,)),
        ],
        compiler_params=pltpu.CompilerParams(needs_layout_passes=False),
    )
    return run(features_table, neighbors.reshape(-1), pick)


# speedup vs baseline: 2.1649x; 1.7932x over previous
import jax
import jax.numpy as jnp
from jax import lax
from jax.experimental import pallas as pl
from jax.experimental.pallas import tpu as pltpu
from jax.experimental.pallas import tpu_sc as plsc

N_NODES = 100000
D_FEAT = 128
B = 100000
K = 16

NC = 2
NS = 16
NW = NC * NS

CHUNK = 3200
SUB = 128
NSUB = CHUNK // SUB
NBUF = 4
LOOKAHEAD = 2
LAST_BASE = B - CHUNK
LAST_ALIGNED = 96768
SLICE = CHUNK + 128


def _body(table_hbm, neigh_hbm, pick_hbm, out_hbm,
          pick_v, neigh_v, chosen_v, rows_v, sem_g, sem_s):
    wid = lax.axis_index("s") * NC + lax.axis_index("c")
    base = jnp.minimum(wid * CHUNK, LAST_BASE)
    stage_base = jnp.minimum(wid * CHUNK, LAST_ALIGNED)
    col_off = base - stage_base

    pltpu.sync_copy(pick_hbm.at[pl.ds(base, CHUNK)], pick_v)
    pltpu.sync_copy(neigh_hbm.at[:, pl.ds(stage_base, SLICE)], neigh_v)

    lane = lax.iota(jnp.int32, 16)

    def resolve(j):
        for c8 in range(SUB // 16):
            i = j * (SUB // 16) + c8
            pick16 = pick_v[pl.ds(i * 16, 16)]
            col16 = col_off + i * 16 + lane
            chosen16 = plsc.load_gather(neigh_v, [pick16, col16])
            chosen_v[pl.ds(i * 16, 16)] = chosen16

    def gather_start(j, slot):
        pltpu.make_async_copy(
            table_hbm.at[chosen_v.at[pl.ds(j * SUB, SUB)]],
            rows_v.at[slot], sem_g.at[slot]
        ).start()

    def gather_wait(j, slot):
        pltpu.make_async_copy(
            table_hbm.at[chosen_v.at[pl.ds(j * SUB, SUB)]],
            rows_v.at[slot], sem_g.at[slot]
        ).wait()

    def store_start(j, slot):
        pltpu.make_async_copy(
            rows_v.at[slot], out_hbm.at[pl.ds(base + j * SUB, SUB), :],
            sem_s.at[slot],
        ).start()

    def store_wait(j, slot):
        pltpu.make_async_copy(
            rows_v.at[slot], out_hbm.at[pl.ds(base + j * SUB, SUB), :],
            sem_s.at[slot],
        ).wait()

    for j in range(LOOKAHEAD):
        resolve(j)
        gather_start(j, j % NBUF)

    def step(j, _):
        slot = lax.rem(j, NBUF)
        ahead = j + LOOKAHEAD
        slot_ahead = lax.rem(ahead, NBUF)

        @pl.when(jnp.logical_and(ahead - NBUF >= 0, ahead < NSUB))
        def _():
            store_wait(ahead - NBUF, slot_ahead)

        @pl.when(ahead < NSUB)
        def _():
            resolve(ahead)
            gather_start(ahead, slot_ahead)

        gather_wait(j, slot)
        store_start(j, slot)
        return ()

    lax.fori_loop(0, NSUB, step, (), unroll=False)

    for j in range(NSUB - NBUF, NSUB):
        store_wait(j, j % NBUF)


@jax.jit
def kernel(features_table, neighbors, pick):
    mesh = plsc.VectorSubcoreMesh(core_axis_name="c", subcore_axis_name="s")
    run = pl.kernel(
        _body,
        out_type=jax.ShapeDtypeStruct((B, D_FEAT), jnp.float32),
        mesh=mesh,
        scratch_types=[
            pltpu.VMEM((CHUNK,), jnp.int32),
            pltpu.VMEM((K, SLICE), jnp.int32),
            pltpu.VMEM((CHUNK,), jnp.int32),
            pltpu.VMEM((NBUF, SUB, D_FEAT), jnp.float32),
            pltpu.SemaphoreType.DMA((NBUF,)),
            pltpu.SemaphoreType.DMA((NBUF,)),
        ],
        compiler_params=pltpu.CompilerParams(needs_layout_passes=False),
    )
    return run(features_table, neighbors.T, pick)


# speedup vs baseline: 2.1742x; 1.0043x over previous
import jax
import jax.numpy as jnp
from jax import lax
from jax.experimental import pallas as pl
from jax.experimental.pallas import tpu as pltpu
from jax.experimental.pallas import tpu_sc as plsc

N_NODES = 100000
D_FEAT = 128
B = 100000
K = 16

NC = 2
NS = 16
NW = NC * NS

CHUNK = 3200
SUB = 128
NSUB = CHUNK // SUB
NBUF = 4
LOOKAHEAD = 3
LAST_BASE = B - CHUNK
LAST_ALIGNED = 96768
SLICE = CHUNK + 128


def _body(table_hbm, neigh_hbm, pick_hbm, out_hbm,
          pick_v, neigh_v, chosen_v, rows_v, sem_g, sem_s):
    wid = lax.axis_index("s") * NC + lax.axis_index("c")
    base = jnp.minimum(wid * CHUNK, LAST_BASE)
    stage_base = jnp.minimum(wid * CHUNK, LAST_ALIGNED)
    col_off = base - stage_base

    pltpu.sync_copy(pick_hbm.at[pl.ds(base, CHUNK)], pick_v)
    pltpu.sync_copy(neigh_hbm.at[:, pl.ds(stage_base, SLICE)], neigh_v)

    lane = lax.iota(jnp.int32, 16)

    def resolve(j):
        for c8 in range(SUB // 16):
            i = j * (SUB // 16) + c8
            pick16 = pick_v[pl.ds(i * 16, 16)]
            col16 = col_off + i * 16 + lane
            chosen16 = plsc.load_gather(neigh_v, [pick16, col16])
            chosen_v[pl.ds(i * 16, 16)] = chosen16

    def gather_start(j, slot):
        pltpu.make_async_copy(
            table_hbm.at[chosen_v.at[pl.ds(j * SUB, SUB)]],
            rows_v.at[slot], sem_g.at[slot]
        ).start()

    def gather_wait(j, slot):
        pltpu.make_async_copy(
            table_hbm.at[chosen_v.at[pl.ds(j * SUB, SUB)]],
            rows_v.at[slot], sem_g.at[slot]
        ).wait()

    def store_start(j, slot):
        pltpu.make_async_copy(
            rows_v.at[slot], out_hbm.at[pl.ds(base + j * SUB, SUB), :],
            sem_s.at[slot],
        ).start()

    def store_wait(j, slot):
        pltpu.make_async_copy(
            rows_v.at[slot], out_hbm.at[pl.ds(base + j * SUB, SUB), :],
            sem_s.at[slot],
        ).wait()

    for j in range(LOOKAHEAD):
        resolve(j)
        gather_start(j, j % NBUF)

    def step(j, _):
        slot = lax.rem(j, NBUF)
        ahead = j + LOOKAHEAD
        slot_ahead = lax.rem(ahead, NBUF)

        @pl.when(jnp.logical_and(ahead - NBUF >= 0, ahead < NSUB))
        def _():
            store_wait(ahead - NBUF, slot_ahead)

        @pl.when(ahead < NSUB)
        def _():
            resolve(ahead)
            gather_start(ahead, slot_ahead)

        gather_wait(j, slot)
        store_start(j, slot)
        return ()

    lax.fori_loop(0, NSUB, step, (), unroll=False)

    for j in range(NSUB - NBUF, NSUB):
        store_wait(j, j % NBUF)


@jax.jit
def kernel(features_table, neighbors, pick):
    mesh = plsc.VectorSubcoreMesh(core_axis_name="c", subcore_axis_name="s")
    run = pl.kernel(
        _body,
        out_type=jax.ShapeDtypeStruct((B, D_FEAT), jnp.float32),
        mesh=mesh,
        scratch_types=[
            pltpu.VMEM((CHUNK,), jnp.int32),
            pltpu.VMEM((K, SLICE), jnp.int32),
            pltpu.VMEM((CHUNK,), jnp.int32),
            pltpu.VMEM((NBUF, SUB, D_FEAT), jnp.float32),
            pltpu.SemaphoreType.DMA((NBUF,)),
            pltpu.SemaphoreType.DMA((NBUF,)),
        ],
        compiler_params=pltpu.CompilerParams(needs_layout_passes=False),
    )
    return run(features_table, neighbors.T, pick)


# speedup vs baseline: 2.1852x; 1.0051x over previous
import jax
import jax.numpy as jnp
from jax import lax
from jax.experimental import pallas as pl
from jax.experimental.pallas import tpu as pltpu
from jax.experimental.pallas import tpu_sc as plsc

N_NODES = 100000
D_FEAT = 128
B = 100000
K = 16

NC = 2
NS = 16
NW = NC * NS

CHUNK = 3200
SUB = 128
NSUB = CHUNK // SUB
NBUF = 4
LOOKAHEAD = 3
LAST_BASE = B - CHUNK
LAST_ALIGNED = 96768
SLICE = CHUNK + 128


def _body(table_hbm, neigh_hbm, pick_hbm, out_hbm,
          pick_v, neigh_v, chosen_v, rows_v, sem_g, sem_s):
    wid = lax.axis_index("s") * NC + lax.axis_index("c")
    base = jnp.minimum(wid * CHUNK, LAST_BASE)
    stage_base = jnp.minimum(wid * CHUNK, LAST_ALIGNED)
    col_off = base - stage_base

    cp_pick = pltpu.make_async_copy(
        pick_hbm.at[pl.ds(base, CHUNK)], pick_v, sem_s.at[0])
    cp_neigh = pltpu.make_async_copy(
        neigh_hbm.at[:, pl.ds(stage_base, SLICE)], neigh_v, sem_s.at[1])
    cp_pick.start()
    cp_neigh.start()
    cp_pick.wait()
    cp_neigh.wait()

    lane = lax.iota(jnp.int32, 16)

    def resolve(j):
        for c8 in range(SUB // 16):
            i = j * (SUB // 16) + c8
            pick16 = pick_v[pl.ds(i * 16, 16)]
            col16 = col_off + i * 16 + lane
            chosen16 = plsc.load_gather(neigh_v, [pick16, col16])
            chosen_v[pl.ds(i * 16, 16)] = chosen16

    def gather_start(j, slot):
        pltpu.make_async_copy(
            table_hbm.at[chosen_v.at[pl.ds(j * SUB, SUB)]],
            rows_v.at[slot], sem_g.at[slot]
        ).start()

    def gather_wait(j, slot):
        pltpu.make_async_copy(
            table_hbm.at[chosen_v.at[pl.ds(j * SUB, SUB)]],
            rows_v.at[slot], sem_g.at[slot]
        ).wait()

    def store_start(j, slot):
        pltpu.make_async_copy(
            rows_v.at[slot], out_hbm.at[pl.ds(base + j * SUB, SUB), :],
            sem_s.at[slot],
        ).start()

    def store_wait(j, slot):
        pltpu.make_async_copy(
            rows_v.at[slot], out_hbm.at[pl.ds(base + j * SUB, SUB), :],
            sem_s.at[slot],
        ).wait()

    for j in range(LOOKAHEAD):
        resolve(j)
        gather_start(j, j % NBUF)

    def step(j, _):
        slot = lax.rem(j, NBUF)
        ahead = j + LOOKAHEAD
        slot_ahead = lax.rem(ahead, NBUF)

        gather_wait(j, slot)
        store_start(j, slot)

        @pl.when(ahead < NSUB)
        def _():
            resolve(ahead)
            @pl.when(ahead - NBUF >= 0)
            def _():
                store_wait(ahead - NBUF, slot_ahead)
            gather_start(ahead, slot_ahead)

        return ()

    lax.fori_loop(0, NSUB, step, (), unroll=False)

    for j in range(NSUB - NBUF, NSUB):
        store_wait(j, j % NBUF)


@jax.jit
def kernel(features_table, neighbors, pick):
    mesh = plsc.VectorSubcoreMesh(core_axis_name="c", subcore_axis_name="s")
    run = pl.kernel(
        _body,
        out_type=jax.ShapeDtypeStruct((B, D_FEAT), jnp.float32),
        mesh=mesh,
        scratch_types=[
            pltpu.VMEM((CHUNK,), jnp.int32),
            pltpu.VMEM((K, SLICE), jnp.int32),
            pltpu.VMEM((CHUNK,), jnp.int32),
            pltpu.VMEM((NBUF, SUB, D_FEAT), jnp.float32),
            pltpu.SemaphoreType.DMA((NBUF,)),
            pltpu.SemaphoreType.DMA((NBUF,)),
        ],
        compiler_params=pltpu.CompilerParams(needs_layout_passes=False),
    )
    return run(features_table, neighbors.T, pick)


# speedup vs baseline: 2.2015x; 1.0075x over previous
import jax
import jax.numpy as jnp
from jax import lax
from jax.experimental import pallas as pl
from jax.experimental.pallas import tpu as pltpu
from jax.experimental.pallas import tpu_sc as plsc

N_NODES = 100000
D_FEAT = 128
B = 100000
K = 16

NC = 2
NS = 16
NW = NC * NS

CHUNK = 3200
SUB = 64
NSUB = CHUNK // SUB
NBUF = 8
LOOKAHEAD = 5
LAST_BASE = B - CHUNK
LAST_ALIGNED = 96768
SLICE = CHUNK + 128


def _body(table_hbm, neigh_hbm, pick_hbm, out_hbm,
          pick_v, neigh_v, chosen_v, rows_v, sem_g, sem_s):
    wid = lax.axis_index("s") * NC + lax.axis_index("c")
    base = jnp.minimum(wid * CHUNK, LAST_BASE)
    stage_base = jnp.minimum(wid * CHUNK, LAST_ALIGNED)
    col_off = base - stage_base

    cp_pick = pltpu.make_async_copy(
        pick_hbm.at[pl.ds(base, CHUNK)], pick_v, sem_s.at[0])
    cp_neigh = pltpu.make_async_copy(
        neigh_hbm.at[:, pl.ds(stage_base, SLICE)], neigh_v, sem_s.at[1])
    cp_pick.start()
    cp_neigh.start()
    cp_pick.wait()
    cp_neigh.wait()

    lane = lax.iota(jnp.int32, 16)

    def resolve(j):
        for c8 in range(SUB // 16):
            i = j * (SUB // 16) + c8
            pick16 = pick_v[pl.ds(i * 16, 16)]
            col16 = col_off + i * 16 + lane
            chosen16 = plsc.load_gather(neigh_v, [pick16, col16])
            chosen_v[pl.ds(i * 16, 16)] = chosen16

    def gather_start(j, slot):
        pltpu.make_async_copy(
            table_hbm.at[chosen_v.at[pl.ds(j * SUB, SUB)]],
            rows_v.at[slot], sem_g.at[slot]
        ).start()

    def gather_wait(j, slot):
        pltpu.make_async_copy(
            table_hbm.at[chosen_v.at[pl.ds(j * SUB, SUB)]],
            rows_v.at[slot], sem_g.at[slot]
        ).wait()

    def store_start(j, slot):
        pltpu.make_async_copy(
            rows_v.at[slot], out_hbm.at[pl.ds(base + j * SUB, SUB), :],
            sem_s.at[slot],
        ).start()

    def store_wait(j, slot):
        pltpu.make_async_copy(
            rows_v.at[slot], out_hbm.at[pl.ds(base + j * SUB, SUB), :],
            sem_s.at[slot],
        ).wait()

    for j in range(LOOKAHEAD):
        resolve(j)
        gather_start(j, j % NBUF)

    def step(j, _):
        slot = lax.rem(j, NBUF)
        ahead = j + LOOKAHEAD
        slot_ahead = lax.rem(ahead, NBUF)

        gather_wait(j, slot)
        store_start(j, slot)

        @pl.when(ahead < NSUB)
        def _():
            resolve(ahead)
            @pl.when(ahead - NBUF >= 0)
            def _():
                store_wait(ahead - NBUF, slot_ahead)
            gather_start(ahead, slot_ahead)

        return ()

    lax.fori_loop(0, NSUB, step, (), unroll=False)

    for j in range(NSUB - NBUF, NSUB):
        store_wait(j, j % NBUF)


@jax.jit
def kernel(features_table, neighbors, pick):
    mesh = plsc.VectorSubcoreMesh(core_axis_name="c", subcore_axis_name="s")
    run = pl.kernel(
        _body,
        out_type=jax.ShapeDtypeStruct((B, D_FEAT), jnp.float32),
        mesh=mesh,
        scratch_types=[
            pltpu.VMEM((CHUNK,), jnp.int32),
            pltpu.VMEM((K, SLICE), jnp.int32),
            pltpu.VMEM((CHUNK,), jnp.int32),
            pltpu.VMEM((NBUF, SUB, D_FEAT), jnp.float32),
            pltpu.SemaphoreType.DMA((NBUF,)),
            pltpu.SemaphoreType.DMA((NBUF,)),
        ],
        compiler_params=pltpu.CompilerParams(needs_layout_passes=False),
    )
    return run(features_table, neighbors.T, pick)
